# baseline (speedup 1.0000x reference)
import functools

import jax
import jax.numpy as jnp
from jax import lax
from jax.experimental import pallas as pl
from jax.experimental.pallas import tpu as pltpu
from jax.experimental.pallas import tpu_sc as plsc

NC = 2
NS = 16
NW = NC * NS
LANES = 16
CHUNK = 128
ZROWS = 128
NBUF = 4


def _sc_agg(x2, src2_t, dst_t, n_pad, with_cnt):
    dh = x2.shape[1]
    num_chunks = src2_t.shape[2]
    rows_per_tile = n_pad // NS
    half_chunks = num_chunks // 2

    mesh = plsc.VectorSubcoreMesh(core_axis_name="c", subcore_axis_name="s")
    out_type = [jax.ShapeDtypeStruct((NC, n_pad, dh), jnp.float32)]
    if with_cnt:
        out_type.append(jax.ShapeDtypeStruct((NC, n_pad, LANES), jnp.float32))

    scratch_types = [
        pltpu.VMEM((num_chunks, CHUNK), jnp.int32),
        pltpu.VMEM((num_chunks, CHUNK), jnp.int32),
        [pltpu.VMEM((CHUNK, dh), jnp.float32)] * NBUF,
        [pltpu.SemaphoreType.DMA] * NBUF,
        [pltpu.SemaphoreType.DMA] * NBUF,
        pltpu.VMEM_SHARED((n_pad, dh), jnp.float32),
    ]
    if with_cnt:
        scratch_types += [
            pltpu.VMEM((CHUNK, LANES), jnp.float32),
            pltpu.VMEM((CHUNK, LANES), jnp.float32),
            pltpu.SemaphoreType.DMA,
            pltpu.VMEM_SHARED((n_pad, LANES), jnp.float32),
        ]

    def body(x_hbm, src_hbm, dst_hbm, *refs):
        csem = None
        if with_cnt:
            (out_hbm, cnt_hbm, src_v, dst_v, rows, gsem, ssem, acc_sh,
             ones_v, zeros_v, csem, cnt_sh) = refs
        else:
            out_hbm, src_v, dst_v, rows, gsem, ssem, acc_sh = refs

        core = lax.axis_index("c")
        tid = lax.axis_index("s")

        @pl.loop(0, CHUNK)
        def _(i):
            @pl.loop(0, dh // LANES)
            def _(j):
                rows[0][i, pl.ds(j * LANES, LANES)] = jnp.zeros(
                    (LANES,), jnp.float32)
            if with_cnt:
                ones_v[i, :] = jnp.ones((LANES,), jnp.float32)
                zeros_v[i, :] = jnp.zeros((LANES,), jnp.float32)

        row0 = tid * rows_per_tile

        @pl.loop(0, rows_per_tile // ZROWS)
        def _(k):
            pltpu.sync_copy(rows[0], acc_sh.at[pl.ds(row0 + k * ZROWS, ZROWS)])
            if with_cnt:
                pltpu.sync_copy(
                    zeros_v, cnt_sh.at[pl.ds(row0 + k * ZROWS, ZROWS)])

        pltpu.sync_copy(src_hbm.at[core, tid], src_v)
        pltpu.sync_copy(dst_hbm.at[tid], dst_v)

        plsc.subcore_barrier()

        def gath(c, b):
            return pltpu.make_async_copy(
                x_hbm.at[src_v.at[c]], rows[b], gsem[b])

        def scat(c, b):
            return pltpu.make_async_copy(
                rows[b], acc_sh.at[dst_v.at[c]], ssem[b])

        def do_cnt(c):
            if with_cnt:
                lo = core * half_chunks
                @pl.when(jnp.logical_and(lo <= c, c < lo + half_chunks))
                def _():
                    pltpu.async_copy(
                        ones_v, cnt_sh.at[dst_v.at[c]], csem, add=True)

        gath(0, 0).start()
        gath(1, 1).start()
        for c in range(2):
            gath(c, c).wait()
            scat(c, c).start(add=True)
            do_cnt(c)
            gath(c + 2, c + 2).start()

        @pl.loop(2, num_chunks - 2, step=NBUF)
        def _(g):
            for i in range(NBUF):
                b = (2 + i) % NBUF
                c = g + i
                scat(c - 2, (b + 2) % NBUF).wait()
                gath(c + 2, (b + 2) % NBUF).start()
                gath(c, b).wait()
                scat(c, b).start(add=True)
                do_cnt(c)

        for c in range(num_chunks - 2, num_chunks):
            b = c % NBUF
            scat(c - 2, (b + 2) % NBUF).wait()
            gath(c, b).wait()
            scat(c, b).start(add=True)
            do_cnt(c)
        for c in range(num_chunks - 2, num_chunks):
            scat(c, c % NBUF).wait()
        if with_cnt:
            @pl.loop(0, half_chunks)
            def _(i):
                pltpu.make_async_copy(
                    ones_v, cnt_sh.at[dst_v.at[0]], csem).wait()

        plsc.subcore_barrier()

        @pl.loop(0, rows_per_tile // ZROWS)
        def _(k):
            r = row0 + k * ZROWS
            pltpu.sync_copy(acc_sh.at[pl.ds(r, ZROWS)],
                            out_hbm.at[core, pl.ds(r, ZROWS)])
            if with_cnt:
                pltpu.sync_copy(cnt_sh.at[pl.ds(r, ZROWS)],
                                cnt_hbm.at[core, pl.ds(r, ZROWS)])

    run = pl.kernel(
        body, out_type=out_type, mesh=mesh, scratch_types=scratch_types,
        compiler_params=pltpu.CompilerParams(use_tc_tiling_on_sc=False))
    return run(x2, src2_t, dst_t)


def _lin_body(x_ref, w_ref, b_ref, o_ref):
    o_ref[...] = lax.dot(
        x_ref[...], w_ref[...], precision=lax.Precision.HIGHEST,
        preferred_element_type=jnp.float32) + b_ref[...]


def _tc_lin(x_pad, w_t, b, block_rows):
    n_pad, d = x_pad.shape
    return pl.pallas_call(
        _lin_body,
        grid=(n_pad // block_rows,),
        in_specs=[
            pl.BlockSpec((block_rows, d), lambda i: (i, 0)),
            pl.BlockSpec((d, d), lambda i: (0, 0)),
            pl.BlockSpec((1, d), lambda i: (0, 0)),
        ],
        out_specs=pl.BlockSpec((block_rows, d), lambda i: (i, 0)),
        out_shape=jax.ShapeDtypeStruct((n_pad, d), jnp.float32),
    )(x_pad, w_t, b.reshape(1, d))


def _combine_body(acc_ref, cnt_ref, w_ref, xr_ref, o_ref, *, relu):
    agg = jnp.concatenate([acc_ref[0], acc_ref[1]], axis=-1)
    cnt = cnt_ref[0, :, 0:1] + cnt_ref[1, :, 0:1]
    agg = agg / jnp.maximum(cnt, 1.0)
    y = lax.dot(agg, w_ref[...], precision=lax.Precision.HIGHEST,
                preferred_element_type=jnp.float32) + xr_ref[...]
    o_ref[...] = jnp.maximum(y, 0.0) if relu else y


def _tc_combine(acc, cnt, w_t, xr, relu, block_rows):
    n_pad, d = xr.shape
    return pl.pallas_call(
        functools.partial(_combine_body, relu=relu),
        grid=(n_pad // block_rows,),
        in_specs=[
            pl.BlockSpec((NC, block_rows, d // 2), lambda i: (0, i, 0)),
            pl.BlockSpec((NC, block_rows, LANES), lambda i: (0, i, 0)),
            pl.BlockSpec((d, d), lambda i: (0, 0)),
            pl.BlockSpec((block_rows, d), lambda i: (i, 0)),
        ],
        out_specs=pl.BlockSpec((block_rows, d), lambda i: (i, 0)),
        out_shape=jax.ShapeDtypeStruct((n_pad, d), jnp.float32),
    )(acc, cnt, w_t, xr)


def kernel(x, edge_index, W1l, b1l, W1r, W2l, b2l, W2r):
    n, d = x.shape
    e = edge_index.shape[1]

    n_pad = ((n + 1 + NS * ZROWS - 1) // (NS * ZROWS)) * (NS * ZROWS)
    num_chunks = (e + NS * CHUNK - 1) // (NS * CHUNK)
    num_chunks = ((num_chunks + NBUF - 1) // NBUF) * NBUF
    e_pad = NS * num_chunks * CHUNK

    src = jnp.concatenate(
        [edge_index[0], jnp.zeros((e_pad - e,), jnp.int32)])
    dst = jnp.concatenate(
        [edge_index[1], jnp.full((e_pad - e,), n, jnp.int32)])
    src2_t = jnp.stack([2 * src, 2 * src + 1]).reshape(
        NC, NS, num_chunks, CHUNK)
    dst_t = dst.reshape(NS, num_chunks, CHUNK)

    x_pad = jnp.zeros((n_pad, d), jnp.float32).at[:n].set(x)

    block_rows = 512

    def to2(a):
        return a.reshape(2 * n_pad, d // 2)

    acc1, cnt = _sc_agg(to2(x_pad), src2_t, dst_t, n_pad, with_cnt=True)
    xr1 = _tc_lin(x_pad, W1r.T, b1l, block_rows)
    h = _tc_combine(acc1, cnt, W1l.T, xr1, relu=True, block_rows=block_rows)

    acc2 = _sc_agg(to2(h), src2_t, dst_t, n_pad, with_cnt=False)[0]
    xr2 = _tc_lin(h, W2r.T, b2l, block_rows)
    out = _tc_combine(acc2, cnt, W2l.T, xr2, relu=False,
                      block_rows=block_rows)
    return out[:n]


# speedup vs baseline: 1.4373x; 1.4373x over previous
import functools

import jax
import jax.numpy as jnp
from jax import lax
from jax.experimental import pallas as pl
from jax.experimental.pallas import tpu as pltpu
from jax.experimental.pallas import tpu_sc as plsc

NC = 2
NS = 16
NW = NC * NS
LANES = 16
CHUNK = 128
ZROWS = 128
NBUF = 2


def _sc_agg(x2, src2_t, dst_t, n_pad, with_cnt):
    dh = x2.shape[1]
    num_chunks = src2_t.shape[2]
    rows_per_tile = n_pad // NS
    half_chunks = num_chunks // 2

    mesh = plsc.VectorSubcoreMesh(core_axis_name="c", subcore_axis_name="s")
    out_type = [jax.ShapeDtypeStruct((NC, n_pad, dh), jnp.float32)]
    if with_cnt:
        out_type.append(jax.ShapeDtypeStruct((NC, n_pad, LANES), jnp.float32))

    scratch_types = [
        pltpu.VMEM((num_chunks, CHUNK), jnp.int32),
        pltpu.VMEM((num_chunks, CHUNK), jnp.int32),
        [pltpu.VMEM((CHUNK, dh), jnp.float32)] * NBUF,
        [pltpu.SemaphoreType.DMA] * NBUF,
        pltpu.VMEM_SHARED((n_pad, dh), jnp.float32),
    ]
    if with_cnt:
        scratch_types += [
            pltpu.VMEM((CHUNK, LANES), jnp.float32),
            pltpu.VMEM((CHUNK, LANES), jnp.float32),
            pltpu.SemaphoreType.DMA,
            pltpu.VMEM_SHARED((n_pad, LANES), jnp.float32),
        ]

    def body(x_hbm, src_hbm, dst_hbm, *refs):
        if with_cnt:
            (out_hbm, cnt_hbm, src_v, dst_v, rows, gsem, acc_sh,
             ones_v, zeros_v, csem, cnt_sh) = refs
        else:
            out_hbm, src_v, dst_v, rows, gsem, acc_sh = refs

        core = lax.axis_index("c")
        tid = lax.axis_index("s")

        @pl.loop(0, CHUNK)
        def _(i):
            @pl.loop(0, dh // LANES)
            def _(j):
                rows[0][i, pl.ds(j * LANES, LANES)] = jnp.zeros(
                    (LANES,), jnp.float32)
            if with_cnt:
                ones_v[i, :] = jnp.ones((LANES,), jnp.float32)
                zeros_v[i, :] = jnp.zeros((LANES,), jnp.float32)

        row0 = tid * rows_per_tile

        @pl.loop(0, rows_per_tile // ZROWS)
        def _(k):
            pltpu.sync_copy(rows[0], acc_sh.at[pl.ds(row0 + k * ZROWS, ZROWS)])
            if with_cnt:
                pltpu.sync_copy(
                    zeros_v, cnt_sh.at[pl.ds(row0 + k * ZROWS, ZROWS)])

        pltpu.sync_copy(src_hbm.at[core, tid], src_v)
        pltpu.sync_copy(dst_hbm.at[tid], dst_v)

        plsc.subcore_barrier()

        def do_cnt(c):
            if with_cnt:
                lo = core * half_chunks
                @pl.when(jnp.logical_and(lo <= c, c < lo + half_chunks))
                def _():
                    pltpu.async_copy(
                        ones_v, cnt_sh.at[dst_v.at[c]], csem, add=True)

        pltpu.async_copy(x_hbm.at[src_v.at[0]], rows[0], gsem[0])

        @pl.loop(0, num_chunks - 2, step=2)
        def _(g):
            for b in range(2):
                c = g + b
                pltpu.async_copy(
                    x_hbm.at[src_v.at[c + 1]], rows[1 - b], gsem[1 - b])
                pltpu.make_async_copy(
                    x_hbm.at[src_v.at[c]], rows[b], gsem[b]).wait()
                pltpu.sync_copy(rows[b], acc_sh.at[dst_v.at[c]], add=True)
                do_cnt(c)

        pltpu.async_copy(
            x_hbm.at[src_v.at[num_chunks - 1]], rows[1], gsem[1])
        pltpu.make_async_copy(
            x_hbm.at[src_v.at[num_chunks - 2]], rows[0], gsem[0]).wait()
        pltpu.sync_copy(rows[0], acc_sh.at[dst_v.at[num_chunks - 2]], add=True)
        do_cnt(num_chunks - 2)
        pltpu.make_async_copy(
            x_hbm.at[src_v.at[num_chunks - 1]], rows[1], gsem[1]).wait()
        pltpu.sync_copy(rows[1], acc_sh.at[dst_v.at[num_chunks - 1]], add=True)
        do_cnt(num_chunks - 1)

        if with_cnt:
            @pl.loop(0, half_chunks)
            def _(i):
                pltpu.make_async_copy(
                    ones_v, cnt_sh.at[dst_v.at[0]], csem).wait()

        plsc.subcore_barrier()

        @pl.loop(0, rows_per_tile // ZROWS)
        def _(k):
            r = row0 + k * ZROWS
            pltpu.sync_copy(acc_sh.at[pl.ds(r, ZROWS)],
                            out_hbm.at[core, pl.ds(r, ZROWS)])
            if with_cnt:
                pltpu.sync_copy(cnt_sh.at[pl.ds(r, ZROWS)],
                                cnt_hbm.at[core, pl.ds(r, ZROWS)])

    run = pl.kernel(
        body, out_type=out_type, mesh=mesh, scratch_types=scratch_types,
        compiler_params=pltpu.CompilerParams(use_tc_tiling_on_sc=False))
    return run(x2, src2_t, dst_t)


def _lin_body(x_ref, w_ref, b_ref, o_ref):
    o_ref[...] = lax.dot(
        x_ref[...], w_ref[...], precision=lax.Precision.HIGHEST,
        preferred_element_type=jnp.float32) + b_ref[...]


def _tc_lin(x_pad, w_t, b, block_rows):
    n_pad, d = x_pad.shape
    return pl.pallas_call(
        _lin_body,
        grid=(n_pad // block_rows,),
        in_specs=[
            pl.BlockSpec((block_rows, d), lambda i: (i, 0)),
            pl.BlockSpec((d, d), lambda i: (0, 0)),
            pl.BlockSpec((1, d), lambda i: (0, 0)),
        ],
        out_specs=pl.BlockSpec((block_rows, d), lambda i: (i, 0)),
        out_shape=jax.ShapeDtypeStruct((n_pad, d), jnp.float32),
    )(x_pad, w_t, b.reshape(1, d))


def _combine_body(acc_ref, cnt_ref, w_ref, xr_ref, o_ref, *, relu):
    agg = jnp.concatenate([acc_ref[0], acc_ref[1]], axis=-1)
    cnt = cnt_ref[0, :, 0:1] + cnt_ref[1, :, 0:1]
    agg = agg / jnp.maximum(cnt, 1.0)
    y = lax.dot(agg, w_ref[...], precision=lax.Precision.HIGHEST,
                preferred_element_type=jnp.float32) + xr_ref[...]
    o_ref[...] = jnp.maximum(y, 0.0) if relu else y


def _tc_combine(acc, cnt, w_t, xr, relu, block_rows):
    n_pad, d = xr.shape
    return pl.pallas_call(
        functools.partial(_combine_body, relu=relu),
        grid=(n_pad // block_rows,),
        in_specs=[
            pl.BlockSpec((NC, block_rows, d // 2), lambda i: (0, i, 0)),
            pl.BlockSpec((NC, block_rows, LANES), lambda i: (0, i, 0)),
            pl.BlockSpec((d, d), lambda i: (0, 0)),
            pl.BlockSpec((block_rows, d), lambda i: (i, 0)),
        ],
        out_specs=pl.BlockSpec((block_rows, d), lambda i: (i, 0)),
        out_shape=jax.ShapeDtypeStruct((n_pad, d), jnp.float32),
    )(acc, cnt, w_t, xr)


def kernel(x, edge_index, W1l, b1l, W1r, W2l, b2l, W2r):
    n, d = x.shape
    e = edge_index.shape[1]

    n_pad = ((n + 1 + NS * ZROWS - 1) // (NS * ZROWS)) * (NS * ZROWS)
    num_chunks = (e + NS * CHUNK - 1) // (NS * CHUNK)
    num_chunks = ((num_chunks + NBUF - 1) // NBUF) * NBUF
    e_pad = NS * num_chunks * CHUNK

    src = jnp.concatenate(
        [edge_index[0], jnp.zeros((e_pad - e,), jnp.int32)])
    dst = jnp.concatenate(
        [edge_index[1], jnp.full((e_pad - e,), n, jnp.int32)])
    src2_t = jnp.stack([2 * src, 2 * src + 1]).reshape(
        NC, NS, num_chunks, CHUNK)
    dst_t = dst.reshape(NS, num_chunks, CHUNK)

    x_pad = jnp.zeros((n_pad, d), jnp.float32).at[:n].set(x)

    block_rows = 512

    def to2(a):
        return a.reshape(2 * n_pad, d // 2)

    acc1, cnt = _sc_agg(to2(x_pad), src2_t, dst_t, n_pad, with_cnt=True)
    xr1 = _tc_lin(x_pad, W1r.T, b1l, block_rows)
    h = _tc_combine(acc1, cnt, W1l.T, xr1, relu=True, block_rows=block_rows)

    acc2 = _sc_agg(to2(h), src2_t, dst_t, n_pad, with_cnt=False)[0]
    xr2 = _tc_lin(h, W2r.T, b2l, block_rows)
    out = _tc_combine(acc2, cnt, W2l.T, xr2, relu=False,
                      block_rows=block_rows)
    return out[:n]


# speedup vs baseline: 1.4890x; 1.0360x over previous
import functools

import jax
import jax.numpy as jnp
from jax import lax
from jax.experimental import pallas as pl
from jax.experimental.pallas import tpu as pltpu
from jax.experimental.pallas import tpu_sc as plsc

NC = 2
NS = 16
NW = NC * NS
LANES = 16
CHUNK = 128
ZROWS = 128
NBUF = 2


def _sc_agg(x2, src2_t, dst_t, n_pad, with_cnt):
    dh = x2.shape[1]
    num_chunks = src2_t.shape[2]
    rows_per_tile = n_pad // NS
    half_chunks = num_chunks // 2

    mesh = plsc.VectorSubcoreMesh(core_axis_name="c", subcore_axis_name="s")
    out_type = [jax.ShapeDtypeStruct((NC, n_pad, dh), jnp.float32)]
    if with_cnt:
        out_type.append(jax.ShapeDtypeStruct((NC, n_pad, LANES), jnp.float32))

    scratch_types = [
        pltpu.VMEM((num_chunks, CHUNK), jnp.int32),
        pltpu.VMEM((num_chunks, CHUNK), jnp.int32),
        [pltpu.VMEM((CHUNK, dh), jnp.float32)] * NBUF,
        [pltpu.SemaphoreType.DMA] * NBUF,
        pltpu.VMEM_SHARED((n_pad, dh), jnp.float32),
    ]
    if with_cnt:
        scratch_types += [
            pltpu.VMEM((CHUNK, LANES), jnp.float32),
            pltpu.VMEM((CHUNK, LANES), jnp.float32),
            pltpu.SemaphoreType.DMA,
            pltpu.VMEM_SHARED((n_pad, LANES), jnp.float32),
        ]

    def body(x_hbm, src_hbm, dst_hbm, *refs):
        if with_cnt:
            (out_hbm, cnt_hbm, src_v, dst_v, rows, gsem, acc_sh,
             ones_v, zeros_v, csem, cnt_sh) = refs
        else:
            out_hbm, src_v, dst_v, rows, gsem, acc_sh = refs

        core = lax.axis_index("c")
        tid = lax.axis_index("s")

        @pl.loop(0, CHUNK)
        def _(i):
            @pl.loop(0, dh // LANES)
            def _(j):
                rows[0][i, pl.ds(j * LANES, LANES)] = jnp.zeros(
                    (LANES,), jnp.float32)
            if with_cnt:
                ones_v[i, :] = jnp.ones((LANES,), jnp.float32)
                zeros_v[i, :] = jnp.zeros((LANES,), jnp.float32)

        row0 = tid * rows_per_tile

        @pl.loop(0, rows_per_tile // ZROWS)
        def _(k):
            pltpu.sync_copy(rows[0], acc_sh.at[pl.ds(row0 + k * ZROWS, ZROWS)])
            if with_cnt:
                pltpu.sync_copy(
                    zeros_v, cnt_sh.at[pl.ds(row0 + k * ZROWS, ZROWS)])

        pltpu.sync_copy(src_hbm.at[core, tid], src_v)
        pltpu.sync_copy(dst_hbm.at[tid], dst_v)

        plsc.subcore_barrier()

        def do_cnt(c):
            if with_cnt:
                lo = core * half_chunks
                @pl.when(jnp.logical_and(lo <= c, c < lo + half_chunks))
                def _():
                    pltpu.async_copy(
                        ones_v, cnt_sh.at[dst_v.at[c]], csem, add=True)

        pltpu.async_copy(x_hbm.at[src_v.at[0]], rows[0], gsem[0])

        @pl.loop(0, num_chunks - 2, step=2)
        def _(g):
            for b in range(2):
                c = g + b
                pltpu.async_copy(
                    x_hbm.at[src_v.at[c + 1]], rows[1 - b], gsem[1 - b])
                pltpu.make_async_copy(
                    x_hbm.at[src_v.at[c]], rows[b], gsem[b]).wait()
                pltpu.sync_copy(rows[b], acc_sh.at[dst_v.at[c]], add=True)
                do_cnt(c)

        pltpu.async_copy(
            x_hbm.at[src_v.at[num_chunks - 1]], rows[1], gsem[1])
        pltpu.make_async_copy(
            x_hbm.at[src_v.at[num_chunks - 2]], rows[0], gsem[0]).wait()
        pltpu.sync_copy(rows[0], acc_sh.at[dst_v.at[num_chunks - 2]], add=True)
        do_cnt(num_chunks - 2)
        pltpu.make_async_copy(
            x_hbm.at[src_v.at[num_chunks - 1]], rows[1], gsem[1]).wait()
        pltpu.sync_copy(rows[1], acc_sh.at[dst_v.at[num_chunks - 1]], add=True)
        do_cnt(num_chunks - 1)

        if with_cnt:
            @pl.loop(0, half_chunks)
            def _(i):
                pltpu.make_async_copy(
                    ones_v, cnt_sh.at[dst_v.at[0]], csem).wait()

        plsc.subcore_barrier()

        @pl.loop(0, rows_per_tile // ZROWS)
        def _(k):
            r = row0 + k * ZROWS
            pltpu.sync_copy(acc_sh.at[pl.ds(r, ZROWS)],
                            out_hbm.at[core, pl.ds(r, ZROWS)])
            if with_cnt:
                pltpu.sync_copy(cnt_sh.at[pl.ds(r, ZROWS)],
                                cnt_hbm.at[core, pl.ds(r, ZROWS)])

    run = pl.kernel(
        body, out_type=out_type, mesh=mesh, scratch_types=scratch_types,
        compiler_params=pltpu.CompilerParams(use_tc_tiling_on_sc=False))
    return run(x2, src2_t, dst_t)


def _lin_body(x_ref, w_ref, b_ref, o_ref):
    xb = jnp.concatenate([x_ref[0], x_ref[1]], axis=-1)
    o_ref[...] = lax.dot(
        xb, w_ref[...], precision=lax.Precision.HIGHEST,
        preferred_element_type=jnp.float32) + b_ref[...]


def _tc_lin(xb, w_t, b, block_rows):
    _, n_pad, dh = xb.shape
    d = 2 * dh
    return pl.pallas_call(
        _lin_body,
        grid=(n_pad // block_rows,),
        in_specs=[
            pl.BlockSpec((NC, block_rows, dh), lambda i: (0, i, 0)),
            pl.BlockSpec((d, d), lambda i: (0, 0)),
            pl.BlockSpec((1, d), lambda i: (0, 0)),
        ],
        out_specs=pl.BlockSpec((block_rows, d), lambda i: (i, 0)),
        out_shape=jax.ShapeDtypeStruct((n_pad, d), jnp.float32),
    )(xb, w_t, b.reshape(1, d))


def _combine_body(acc_ref, cnt_ref, w_ref, xr_ref, o_ref, *, relu, blocked):
    agg = jnp.concatenate([acc_ref[0], acc_ref[1]], axis=-1)
    cnt = cnt_ref[0, :, 0:1] + cnt_ref[1, :, 0:1]
    agg = agg / jnp.maximum(cnt, 1.0)
    y = lax.dot(agg, w_ref[...], precision=lax.Precision.HIGHEST,
                preferred_element_type=jnp.float32) + xr_ref[...]
    y = jnp.maximum(y, 0.0) if relu else y
    if blocked:
        dh = y.shape[-1] // 2
        o_ref[0] = y[:, :dh]
        o_ref[1] = y[:, dh:]
    else:
        o_ref[...] = y


def _tc_combine(acc, cnt, w_t, xr, relu, block_rows, blocked):
    n_pad, d = xr.shape
    if blocked:
        out_shape = jax.ShapeDtypeStruct((NC, n_pad, d // 2), jnp.float32)
        out_specs = pl.BlockSpec((NC, block_rows, d // 2), lambda i: (0, i, 0))
    else:
        out_shape = jax.ShapeDtypeStruct((n_pad, d), jnp.float32)
        out_specs = pl.BlockSpec((block_rows, d), lambda i: (i, 0))
    return pl.pallas_call(
        functools.partial(_combine_body, relu=relu, blocked=blocked),
        grid=(n_pad // block_rows,),
        in_specs=[
            pl.BlockSpec((NC, block_rows, d // 2), lambda i: (0, i, 0)),
            pl.BlockSpec((NC, block_rows, LANES), lambda i: (0, i, 0)),
            pl.BlockSpec((d, d), lambda i: (0, 0)),
            pl.BlockSpec((block_rows, d), lambda i: (i, 0)),
        ],
        out_specs=out_specs,
        out_shape=out_shape,
    )(acc, cnt, w_t, xr)


def kernel(x, edge_index, W1l, b1l, W1r, W2l, b2l, W2r):
    n, d = x.shape
    dh = d // 2
    e = edge_index.shape[1]

    n_pad = ((n + 1 + NS * ZROWS - 1) // (NS * ZROWS)) * (NS * ZROWS)
    num_chunks = (e + NS * CHUNK - 1) // (NS * CHUNK)
    num_chunks = ((num_chunks + NBUF - 1) // NBUF) * NBUF
    e_pad = NS * num_chunks * CHUNK

    src = jnp.concatenate(
        [edge_index[0], jnp.zeros((e_pad - e,), jnp.int32)])
    dst = jnp.concatenate(
        [edge_index[1], jnp.full((e_pad - e,), n, jnp.int32)])
    src2_t = jnp.stack([src, src + n_pad]).reshape(
        NC, NS, num_chunks, CHUNK)
    dst_t = dst.reshape(NS, num_chunks, CHUNK)

    xb = jnp.zeros((NC, n_pad, dh), jnp.float32)
    xb = xb.at[0, :n].set(x[:, :dh]).at[1, :n].set(x[:, dh:])

    block_rows = 512

    acc1, cnt = _sc_agg(
        xb.reshape(2 * n_pad, dh), src2_t, dst_t, n_pad, with_cnt=True)
    xr1 = _tc_lin(xb, W1r.T, b1l, block_rows)
    hb = _tc_combine(acc1, cnt, W1l.T, xr1, relu=True,
                     block_rows=block_rows, blocked=True)

    acc2 = _sc_agg(
        hb.reshape(2 * n_pad, dh), src2_t, dst_t, n_pad, with_cnt=False)[0]
    xr2 = _tc_lin(hb, W2r.T, b2l, block_rows)
    out = _tc_combine(acc2, cnt, W2l.T, xr2, relu=False,
                      block_rows=block_rows, blocked=False)
    return out[:n]


# speedup vs baseline: 1.4933x; 1.0029x over previous
import functools

import jax
import jax.numpy as jnp
from jax import lax
from jax.experimental import pallas as pl
from jax.experimental.pallas import tpu as pltpu
from jax.experimental.pallas import tpu_sc as plsc

NC = 2
NS = 16
NW = NC * NS
LANES = 16
CHUNK = 128
ZROWS = 128
NBUF = 2


def _sc_agg(x2, src2_t, dst_t, n_pad, with_cnt):
    dh = x2.shape[1]
    num_chunks = src2_t.shape[2]
    rows_per_tile = n_pad // NS
    half_chunks = num_chunks // 2

    mesh = plsc.VectorSubcoreMesh(core_axis_name="c", subcore_axis_name="s")
    out_type = [jax.ShapeDtypeStruct((NC, n_pad, dh), jnp.float32)]
    if with_cnt:
        out_type.append(jax.ShapeDtypeStruct((NC, n_pad, LANES), jnp.float32))

    scratch_types = [
        pltpu.VMEM((num_chunks, CHUNK), jnp.int32),
        pltpu.VMEM((num_chunks, CHUNK), jnp.int32),
        [pltpu.VMEM((CHUNK, dh), jnp.float32)] * NBUF,
        [pltpu.SemaphoreType.DMA] * NBUF,
        pltpu.VMEM_SHARED((n_pad, dh), jnp.float32),
    ]
    if with_cnt:
        scratch_types += [
            pltpu.VMEM((CHUNK, LANES), jnp.float32),
            pltpu.VMEM((CHUNK, LANES), jnp.float32),
            pltpu.SemaphoreType.DMA,
            pltpu.VMEM_SHARED((n_pad, LANES), jnp.float32),
        ]

    def body(x_hbm, src_hbm, dst_hbm, *refs):
        if with_cnt:
            (out_hbm, cnt_hbm, src_v, dst_v, rows, gsem, acc_sh,
             ones_v, zeros_v, csem, cnt_sh) = refs
        else:
            out_hbm, src_v, dst_v, rows, gsem, acc_sh = refs

        core = lax.axis_index("c")
        tid = lax.axis_index("s")

        @pl.loop(0, CHUNK)
        def _(i):
            @pl.loop(0, dh // LANES)
            def _(j):
                rows[0][i, pl.ds(j * LANES, LANES)] = jnp.zeros(
                    (LANES,), jnp.float32)
            if with_cnt:
                ones_v[i, :] = jnp.ones((LANES,), jnp.float32)
                zeros_v[i, :] = jnp.zeros((LANES,), jnp.float32)

        row0 = tid * rows_per_tile

        @pl.loop(0, rows_per_tile // ZROWS)
        def _(k):
            pltpu.sync_copy(rows[0], acc_sh.at[pl.ds(row0 + k * ZROWS, ZROWS)])
            if with_cnt:
                pltpu.sync_copy(
                    zeros_v, cnt_sh.at[pl.ds(row0 + k * ZROWS, ZROWS)])

        pltpu.sync_copy(src_hbm.at[core, tid], src_v)
        pltpu.sync_copy(dst_hbm.at[tid], dst_v)

        plsc.subcore_barrier()

        def do_cnt(c):
            if with_cnt:
                lo = core * half_chunks
                @pl.when(jnp.logical_and(lo <= c, c < lo + half_chunks))
                def _():
                    pltpu.async_copy(
                        ones_v, cnt_sh.at[dst_v.at[c]], csem, add=True)

        pltpu.async_copy(x_hbm.at[src_v.at[0]], rows[0], gsem[0])

        @pl.loop(0, num_chunks - 2, step=2)
        def _(g):
            for b in range(2):
                c = g + b
                pltpu.async_copy(
                    x_hbm.at[src_v.at[c + 1]], rows[1 - b], gsem[1 - b])
                pltpu.make_async_copy(
                    x_hbm.at[src_v.at[c]], rows[b], gsem[b]).wait()
                pltpu.sync_copy(rows[b], acc_sh.at[dst_v.at[c]], add=True)
                do_cnt(c)

        pltpu.async_copy(
            x_hbm.at[src_v.at[num_chunks - 1]], rows[1], gsem[1])
        pltpu.make_async_copy(
            x_hbm.at[src_v.at[num_chunks - 2]], rows[0], gsem[0]).wait()
        pltpu.sync_copy(rows[0], acc_sh.at[dst_v.at[num_chunks - 2]], add=True)
        do_cnt(num_chunks - 2)
        pltpu.make_async_copy(
            x_hbm.at[src_v.at[num_chunks - 1]], rows[1], gsem[1]).wait()
        pltpu.sync_copy(rows[1], acc_sh.at[dst_v.at[num_chunks - 1]], add=True)
        do_cnt(num_chunks - 1)

        if with_cnt:
            @pl.loop(0, half_chunks)
            def _(i):
                pltpu.make_async_copy(
                    ones_v, cnt_sh.at[dst_v.at[0]], csem).wait()

        plsc.subcore_barrier()

        @pl.loop(0, rows_per_tile // ZROWS)
        def _(k):
            r = row0 + k * ZROWS
            pltpu.sync_copy(acc_sh.at[pl.ds(r, ZROWS)],
                            out_hbm.at[core, pl.ds(r, ZROWS)])
            if with_cnt:
                pltpu.sync_copy(cnt_sh.at[pl.ds(r, ZROWS)],
                                cnt_hbm.at[core, pl.ds(r, ZROWS)])

    run = pl.kernel(
        body, out_type=out_type, mesh=mesh, scratch_types=scratch_types,
        compiler_params=pltpu.CompilerParams(use_tc_tiling_on_sc=False))
    return run(x2, src2_t, dst_t)


def _lin_body(x_ref, w_ref, b_ref, o_ref):
    xb = jnp.concatenate([x_ref[0], x_ref[1]], axis=-1)
    o_ref[...] = lax.dot(
        xb, w_ref[...], precision=lax.Precision.DEFAULT,
        preferred_element_type=jnp.float32) + b_ref[...]


def _tc_lin(xb, w_t, b, block_rows):
    _, n_pad, dh = xb.shape
    d = 2 * dh
    return pl.pallas_call(
        _lin_body,
        grid=(n_pad // block_rows,),
        in_specs=[
            pl.BlockSpec((NC, block_rows, dh), lambda i: (0, i, 0)),
            pl.BlockSpec((d, d), lambda i: (0, 0)),
            pl.BlockSpec((1, d), lambda i: (0, 0)),
        ],
        out_specs=pl.BlockSpec((block_rows, d), lambda i: (i, 0)),
        out_shape=jax.ShapeDtypeStruct((n_pad, d), jnp.float32),
    )(xb, w_t, b.reshape(1, d))


def _combine_body(acc_ref, cnt_ref, w_ref, xr_ref, o_ref, *, relu, blocked):
    agg = jnp.concatenate([acc_ref[0], acc_ref[1]], axis=-1)
    cnt = cnt_ref[0, :, 0:1] + cnt_ref[1, :, 0:1]
    agg = agg / jnp.maximum(cnt, 1.0)
    y = lax.dot(agg, w_ref[...], precision=lax.Precision.DEFAULT,
                preferred_element_type=jnp.float32) + xr_ref[...]
    y = jnp.maximum(y, 0.0) if relu else y
    if blocked:
        dh = y.shape[-1] // 2
        o_ref[0] = y[:, :dh]
        o_ref[1] = y[:, dh:]
    else:
        o_ref[...] = y


def _tc_combine(acc, cnt, w_t, xr, relu, block_rows, blocked):
    n_pad, d = xr.shape
    if blocked:
        out_shape = jax.ShapeDtypeStruct((NC, n_pad, d // 2), jnp.float32)
        out_specs = pl.BlockSpec((NC, block_rows, d // 2), lambda i: (0, i, 0))
    else:
        out_shape = jax.ShapeDtypeStruct((n_pad, d), jnp.float32)
        out_specs = pl.BlockSpec((block_rows, d), lambda i: (i, 0))
    return pl.pallas_call(
        functools.partial(_combine_body, relu=relu, blocked=blocked),
        grid=(n_pad // block_rows,),
        in_specs=[
            pl.BlockSpec((NC, block_rows, d // 2), lambda i: (0, i, 0)),
            pl.BlockSpec((NC, block_rows, LANES), lambda i: (0, i, 0)),
            pl.BlockSpec((d, d), lambda i: (0, 0)),
            pl.BlockSpec((block_rows, d), lambda i: (i, 0)),
        ],
        out_specs=out_specs,
        out_shape=out_shape,
    )(acc, cnt, w_t, xr)


def kernel(x, edge_index, W1l, b1l, W1r, W2l, b2l, W2r):
    n, d = x.shape
    dh = d // 2
    e = edge_index.shape[1]

    n_pad = ((n + 1 + NS * ZROWS - 1) // (NS * ZROWS)) * (NS * ZROWS)
    num_chunks = (e + NS * CHUNK - 1) // (NS * CHUNK)
    num_chunks = ((num_chunks + NBUF - 1) // NBUF) * NBUF
    e_pad = NS * num_chunks * CHUNK

    src = jnp.concatenate(
        [edge_index[0], jnp.zeros((e_pad - e,), jnp.int32)])
    dst = jnp.concatenate(
        [edge_index[1], jnp.full((e_pad - e,), n, jnp.int32)])
    src2_t = jnp.stack([src, src + n_pad]).reshape(
        NC, NS, num_chunks, CHUNK)
    dst_t = dst.reshape(NS, num_chunks, CHUNK)

    xb = jnp.zeros((NC, n_pad, dh), jnp.float32)
    xb = xb.at[0, :n].set(x[:, :dh]).at[1, :n].set(x[:, dh:])

    block_rows = 512

    acc1, cnt = _sc_agg(
        xb.reshape(2 * n_pad, dh), src2_t, dst_t, n_pad, with_cnt=True)
    xr1 = _tc_lin(xb, W1r.T, b1l, block_rows)
    hb = _tc_combine(acc1, cnt, W1l.T, xr1, relu=True,
                     block_rows=block_rows, blocked=True)

    acc2 = _sc_agg(
        hb.reshape(2 * n_pad, dh), src2_t, dst_t, n_pad, with_cnt=False)[0]
    xr2 = _tc_lin(hb, W2r.T, b2l, block_rows)
    out = _tc_combine(acc2, cnt, W2l.T, xr2, relu=False,
                      block_rows=block_rows, blocked=False)
    return out[:n]


# speedup vs baseline: 1.5079x; 1.0097x over previous
import functools

import jax
import jax.numpy as jnp
from jax import lax
from jax.experimental import pallas as pl
from jax.experimental.pallas import tpu as pltpu
from jax.experimental.pallas import tpu_sc as plsc

NC = 2
NS = 16
NW = NC * NS
LANES = 16
CHUNK = 128
ZROWS = 128
NBUF = 2


def _sc_agg(x2, src2_t, dst_t, n_pad, with_cnt):
    dh = x2.shape[1]
    num_chunks = src2_t.shape[2]
    rows_per_tile = n_pad // NS
    half_chunks = num_chunks // 2

    mesh = plsc.VectorSubcoreMesh(core_axis_name="c", subcore_axis_name="s")
    out_type = [jax.ShapeDtypeStruct((NC, n_pad, dh), jnp.float32)]
    if with_cnt:
        out_type.append(jax.ShapeDtypeStruct((NC, n_pad, LANES), jnp.float32))

    scratch_types = [
        pltpu.VMEM((num_chunks, CHUNK), jnp.int32),
        pltpu.VMEM((num_chunks, CHUNK), jnp.int32),
        [pltpu.VMEM((CHUNK, dh), jnp.float32)] * NBUF,
        [pltpu.SemaphoreType.DMA] * NBUF,
        pltpu.VMEM_SHARED((n_pad, dh), jnp.float32),
    ]
    if with_cnt:
        scratch_types += [
            pltpu.VMEM((CHUNK, LANES), jnp.float32),
            pltpu.VMEM((CHUNK, LANES), jnp.float32),
            pltpu.SemaphoreType.DMA,
            pltpu.VMEM_SHARED((n_pad, LANES), jnp.float32),
        ]

    def body(x_hbm, src_hbm, dst_hbm, *refs):
        if with_cnt:
            (out_hbm, cnt_hbm, src_v, dst_v, rows, gsem, acc_sh,
             ones_v, zeros_v, csem, cnt_sh) = refs
        else:
            out_hbm, src_v, dst_v, rows, gsem, acc_sh = refs

        core = lax.axis_index("c")
        tid = lax.axis_index("s")

        @pl.loop(0, CHUNK)
        def _(i):
            @pl.loop(0, dh // LANES)
            def _(j):
                rows[0][i, pl.ds(j * LANES, LANES)] = jnp.zeros(
                    (LANES,), jnp.float32)
            if with_cnt:
                ones_v[i, :] = jnp.ones((LANES,), jnp.float32)
                zeros_v[i, :] = jnp.zeros((LANES,), jnp.float32)

        idx_copies = (pltpu.make_async_copy(src_hbm.at[core, tid], src_v,
                                            gsem[0]),
                      pltpu.make_async_copy(dst_hbm.at[tid], dst_v, gsem[1]))
        for cp in idx_copies:
            cp.start()

        row0 = tid * rows_per_tile

        @pl.loop(0, rows_per_tile // ZROWS)
        def _(k):
            pltpu.sync_copy(rows[0], acc_sh.at[pl.ds(row0 + k * ZROWS, ZROWS)])
            if with_cnt:
                pltpu.sync_copy(
                    zeros_v, cnt_sh.at[pl.ds(row0 + k * ZROWS, ZROWS)])

        for cp in idx_copies:
            cp.wait()

        plsc.subcore_barrier()

        def do_cnt(c):
            if with_cnt:
                lo = core * half_chunks
                @pl.when(jnp.logical_and(lo <= c, c < lo + half_chunks))
                def _():
                    pltpu.async_copy(
                        ones_v, cnt_sh.at[dst_v.at[c]], csem, add=True)

        pltpu.async_copy(x_hbm.at[src_v.at[0]], rows[0], gsem[0])

        @pl.loop(0, num_chunks - 2, step=2)
        def _(g):
            for b in range(2):
                c = g + b
                pltpu.async_copy(
                    x_hbm.at[src_v.at[c + 1]], rows[1 - b], gsem[1 - b])
                pltpu.make_async_copy(
                    x_hbm.at[src_v.at[c]], rows[b], gsem[b]).wait()
                pltpu.sync_copy(rows[b], acc_sh.at[dst_v.at[c]], add=True)
                do_cnt(c)

        pltpu.async_copy(
            x_hbm.at[src_v.at[num_chunks - 1]], rows[1], gsem[1])
        pltpu.make_async_copy(
            x_hbm.at[src_v.at[num_chunks - 2]], rows[0], gsem[0]).wait()
        pltpu.sync_copy(rows[0], acc_sh.at[dst_v.at[num_chunks - 2]], add=True)
        do_cnt(num_chunks - 2)
        pltpu.make_async_copy(
            x_hbm.at[src_v.at[num_chunks - 1]], rows[1], gsem[1]).wait()
        pltpu.sync_copy(rows[1], acc_sh.at[dst_v.at[num_chunks - 1]], add=True)
        do_cnt(num_chunks - 1)

        if with_cnt:
            @pl.loop(0, half_chunks)
            def _(i):
                pltpu.make_async_copy(
                    ones_v, cnt_sh.at[dst_v.at[0]], csem).wait()

        plsc.subcore_barrier()

        @pl.loop(0, rows_per_tile // ZROWS)
        def _(k):
            r = row0 + k * ZROWS
            pltpu.sync_copy(acc_sh.at[pl.ds(r, ZROWS)],
                            out_hbm.at[core, pl.ds(r, ZROWS)])
            if with_cnt:
                pltpu.sync_copy(cnt_sh.at[pl.ds(r, ZROWS)],
                                cnt_hbm.at[core, pl.ds(r, ZROWS)])

    run = pl.kernel(
        body, out_type=out_type, mesh=mesh, scratch_types=scratch_types,
        compiler_params=pltpu.CompilerParams(use_tc_tiling_on_sc=False))
    return run(x2, src2_t, dst_t)


def _lin_body(x_ref, w_ref, b_ref, o_ref):
    xb = jnp.concatenate([x_ref[0], x_ref[1]], axis=-1)
    o_ref[...] = lax.dot(
        xb, w_ref[...], precision=lax.Precision.DEFAULT,
        preferred_element_type=jnp.float32) + b_ref[...]


def _tc_lin(xb, w_t, b, block_rows):
    _, n_pad, dh = xb.shape
    d = 2 * dh
    return pl.pallas_call(
        _lin_body,
        grid=(n_pad // block_rows,),
        in_specs=[
            pl.BlockSpec((NC, block_rows, dh), lambda i: (0, i, 0)),
            pl.BlockSpec((d, d), lambda i: (0, 0)),
            pl.BlockSpec((1, d), lambda i: (0, 0)),
        ],
        out_specs=pl.BlockSpec((block_rows, d), lambda i: (i, 0)),
        out_shape=jax.ShapeDtypeStruct((n_pad, d), jnp.float32),
    )(xb, w_t, b.reshape(1, d))


def _combine_body(acc_ref, cnt_ref, w_ref, xr_ref, o_ref, *, relu, blocked):
    agg = jnp.concatenate([acc_ref[0], acc_ref[1]], axis=-1)
    cnt = cnt_ref[0, :, 0:1] + cnt_ref[1, :, 0:1]
    agg = agg / jnp.maximum(cnt, 1.0)
    y = lax.dot(agg, w_ref[...], precision=lax.Precision.DEFAULT,
                preferred_element_type=jnp.float32) + xr_ref[...]
    y = jnp.maximum(y, 0.0) if relu else y
    if blocked:
        dh = y.shape[-1] // 2
        o_ref[0] = y[:, :dh]
        o_ref[1] = y[:, dh:]
    else:
        o_ref[...] = y


def _tc_combine(acc, cnt, w_t, xr, relu, block_rows, blocked):
    n_pad, d = xr.shape
    if blocked:
        out_shape = jax.ShapeDtypeStruct((NC, n_pad, d // 2), jnp.float32)
        out_specs = pl.BlockSpec((NC, block_rows, d // 2), lambda i: (0, i, 0))
    else:
        out_shape = jax.ShapeDtypeStruct((n_pad, d), jnp.float32)
        out_specs = pl.BlockSpec((block_rows, d), lambda i: (i, 0))
    return pl.pallas_call(
        functools.partial(_combine_body, relu=relu, blocked=blocked),
        grid=(n_pad // block_rows,),
        in_specs=[
            pl.BlockSpec((NC, block_rows, d // 2), lambda i: (0, i, 0)),
            pl.BlockSpec((NC, block_rows, LANES), lambda i: (0, i, 0)),
            pl.BlockSpec((d, d), lambda i: (0, 0)),
            pl.BlockSpec((block_rows, d), lambda i: (i, 0)),
        ],
        out_specs=out_specs,
        out_shape=out_shape,
    )(acc, cnt, w_t, xr)


def kernel(x, edge_index, W1l, b1l, W1r, W2l, b2l, W2r):
    n, d = x.shape
    dh = d // 2
    e = edge_index.shape[1]

    n_pad = ((n + 1 + NS * ZROWS - 1) // (NS * ZROWS)) * (NS * ZROWS)
    num_chunks = (e + NS * CHUNK - 1) // (NS * CHUNK)
    num_chunks = ((num_chunks + NBUF - 1) // NBUF) * NBUF
    e_pad = NS * num_chunks * CHUNK

    src = jnp.concatenate(
        [edge_index[0], jnp.zeros((e_pad - e,), jnp.int32)])
    dst = jnp.concatenate(
        [edge_index[1], jnp.full((e_pad - e,), n, jnp.int32)])
    src2_t = jnp.stack([src, src + n_pad]).reshape(
        NC, NS, num_chunks, CHUNK)
    dst_t = dst.reshape(NS, num_chunks, CHUNK)

    xb = jnp.zeros((NC, n_pad, dh), jnp.float32)
    xb = xb.at[0, :n].set(x[:, :dh]).at[1, :n].set(x[:, dh:])

    block_rows = 512

    acc1, cnt = _sc_agg(
        xb.reshape(2 * n_pad, dh), src2_t, dst_t, n_pad, with_cnt=True)
    xr1 = _tc_lin(xb, W1r.T, b1l, block_rows)
    hb = _tc_combine(acc1, cnt, W1l.T, xr1, relu=True,
                     block_rows=block_rows, blocked=True)

    acc2 = _sc_agg(
        hb.reshape(2 * n_pad, dh), src2_t, dst_t, n_pad, with_cnt=False)[0]
    xr2 = _tc_lin(hb, W2r.T, b2l, block_rows)
    out = _tc_combine(acc2, cnt, W2l.T, xr2, relu=False,
                      block_rows=block_rows, blocked=False)
    return out[:n]
